# baseline (speedup 1.0000x reference)
import functools
import math

import jax
import jax.numpy as jnp
from jax import lax
from jax.experimental import pallas as pl
from jax.experimental.pallas import tpu as pltpu
from jax.experimental.pallas import tpu_sc as plsc

EMBED = 128
NREL = 16
NNODES = 10000
NEDGES = 320000
L = 16
BLK = 128
NBLK = NEDGES // BLK
NC, NS = 2, 16
NW = NC * NS
KMAX = (NBLK + NW - 1) // NW
R = 3

_mesh = plsc.VectorSubcoreMesh(
    core_axis_name="c", subcore_axis_name="s", num_cores=NC, num_subcores=NS
)


@functools.partial(
    pl.kernel,
    out_type=jax.ShapeDtypeStruct((NEDGES,), jnp.float32),
    mesh=_mesh,
    scratch_types=[
        [pltpu.VMEM((3, BLK), jnp.int32) for _ in range(R)],
        [pltpu.VMEM((BLK, EMBED), jnp.float32) for _ in range(R)],
        [pltpu.VMEM((BLK,), jnp.float32) for _ in range(R)],
        pltpu.VMEM((NREL * EMBED,), jnp.float32),
        pltpu.VMEM((BLK * L,), jnp.float32),
        pltpu.VMEM((2 * L,), jnp.float32),
        [pltpu.SemaphoreType.DMA for _ in range(R)],
        [pltpu.SemaphoreType.DMA for _ in range(R)],
        [pltpu.SemaphoreType.DMA for _ in range(R)],
        [pltpu.SemaphoreType.DMA for _ in range(R)],
    ],
    compiler_params=pltpu.CompilerParams(needs_layout_passes=False),
)
def _sc_scores(table, idx3, rel_flat, params, out,
               idx_v, urows, out_v, rel_v, accbuf, par_v,
               semi, semh, sema, semo):
    wid = lax.axis_index("s") * NC + lax.axis_index("c")

    pltpu.sync_copy(rel_flat, rel_v)
    pltpu.sync_copy(params, par_v)
    scale = par_v[pl.ds(0, L)]
    off = par_v[pl.ds(L, L)]
    lanes = lax.iota(jnp.int32, L)

    def jof(k):
        return wid + NW * k

    def valid(k):
        return (k >= 0) & (jof(k) < NBLK)

    def issue_idx(k, b):
        @pl.when(valid(k))
        def _():
            pltpu.async_copy(idx3.at[jof(k)], idx_v[b], semi[b])

    def drain_idx(k, b):
        @pl.when(valid(k))
        def _():
            pltpu.make_async_copy(idx3.at[0], idx_v[b], semi[b]).wait()

    def issue_h(k, b):
        @pl.when(valid(k))
        def _():
            pltpu.async_copy(table.at[idx_v[b].at[0]], urows[b], semh[b])

    def drain_h(k, b):
        @pl.when(valid(k))
        def _():
            pltpu.make_async_copy(
                table.at[pl.ds(0, BLK)], urows[b], semh[b]).wait()

    def issue_adds(k, b):
        @pl.when(valid(k))
        def _():
            pltpu.async_copy(table.at[idx_v[b].at[1]], urows[b], sema[b],
                             add=True)

    def drain_adds(k, b):
        @pl.when(valid(k))
        def _():
            pltpu.make_async_copy(
                table.at[pl.ds(0, BLK)], urows[b], sema[b]).wait()

    def drain_out(k, b):
        @pl.when(valid(k))
        def _():
            pltpu.make_async_copy(
                out_v[b], out.at[pl.ds(jof(k) * BLK, BLK)], semo[b]).wait()

    def compute(k, b):
        @pl.when(valid(k))
        def _():
            u = urows[b]
            ridx = idx_v[b]

            def grp1(g, c2):
                rb = ridx[2, pl.ds(g * L, L)] * EMBED
                for i in range(L):
                    e = g * L + i
                    rbase = rb[i]
                    acc0 = jnp.zeros((L,), jnp.float32)
                    acc1 = jnp.zeros((L,), jnp.float32)
                    for c in range(EMBED // L):
                        v = (u[e, pl.ds(c * L, L)]
                             + rel_v[pl.ds(rbase + c * L, L)])
                        if c % 2 == 0:
                            acc0 = acc0 + v * v
                        else:
                            acc1 = acc1 + v * v
                    accbuf[pl.ds(e * L, L)] = acc0 + acc1
                return c2

            lax.fori_loop(0, BLK // L, grp1, 0)

            def grp2(g, c2):
                ebase = (g * L + lanes) * L
                s0 = plsc.load_gather(accbuf, [ebase])
                s1 = plsc.load_gather(accbuf, [ebase + 1])
                for c in range(2, L, 2):
                    s0 = s0 + plsc.load_gather(accbuf, [ebase + c])
                    s1 = s1 + plsc.load_gather(accbuf, [ebase + c + 1])
                x = s0 + s1
                i = lax.bitcast_convert_type(x, jnp.int32)
                z = lax.bitcast_convert_type(
                    jnp.int32(0x5F3759DF) - lax.shift_right_logical(i, 1),
                    jnp.float32,
                )
                hx = 0.5 * x
                for _ in range(2):
                    z = z * (1.5 - hx * z * z)
                y = x * z
                out_v[b][pl.ds(g * L, L)] = off - scale * y
                return c2

            lax.fori_loop(0, BLK // L, grp2, 0)
            pltpu.async_copy(out_v[b], out.at[pl.ds(jof(k) * BLK, BLK)],
                             semo[b])

    def step(k, b):
        b1, b2 = (b + 1) % R, (b + 2) % R
        drain_out(k - 2, b1)
        drain_adds(k, b)
        drain_h(k + 1, b1)
        issue_adds(k + 1, b1)
        drain_idx(k + 2, b2)
        issue_h(k + 2, b2)
        compute(k, b)
        issue_idx(k + 3, b)

    issue_idx(0, 0)
    drain_idx(0, 0)
    issue_h(0, 0)
    drain_h(0, 0)
    issue_adds(0, 0)
    issue_idx(1, 1)
    drain_idx(1, 1)
    issue_h(1, 1)
    issue_idx(2, 2)

    ntriples = (KMAX - 1) // R

    def triple(t, carry):
        k = R * t
        step(k, 0)
        step(k + 1, 1)
        step(k + 2, 2)
        return carry

    lax.fori_loop(0, ntriples, triple, 0)
    for k in range(R * ntriples, KMAX):
        step(k, k % R)
    drain_out(KMAX - 2, (KMAX - 2) % R)
    drain_out(KMAX - 1, (KMAX - 1) % R)


def kernel(node_embeddings, edge_index, relation_type, rel_emb, temperature, bias):
    table = jnp.concatenate([node_embeddings, -node_embeddings], axis=0)
    hidx = edge_index[0].astype(jnp.int32)
    tidx = edge_index[1].astype(jnp.int32) + NNODES
    ridx = relation_type.astype(jnp.int32)
    idx3 = jnp.stack(
        [hidx.reshape(NBLK, BLK), tidx.reshape(NBLK, BLK),
         ridx.reshape(NBLK, BLK)], axis=1)
    scale = (1.0 / (temperature * math.sqrt(EMBED))).astype(jnp.float32)
    off = (bias / temperature).astype(jnp.float32)
    params = jnp.concatenate(
        [jnp.broadcast_to(scale, (L,)), jnp.broadcast_to(off, (L,))]
    )
    return _sc_scores(table, idx3, rel_emb.reshape(-1), params)


# speedup vs baseline: 1.1759x; 1.1759x over previous
import functools
import math

import jax
import jax.numpy as jnp
from jax import lax
from jax.experimental import pallas as pl
from jax.experimental.pallas import tpu as pltpu
from jax.experimental.pallas import tpu_sc as plsc

EMBED = 128
NREL = 16
NNODES = 10000
NEDGES = 320000
L = 16
BLK = 128
NBLK = NEDGES // BLK
NC, NS = 2, 16
NW = NC * NS
KMAX = (NBLK + NW - 1) // NW
R = 3

_mesh = plsc.VectorSubcoreMesh(
    core_axis_name="c", subcore_axis_name="s", num_cores=NC, num_subcores=NS
)


@functools.partial(
    pl.kernel,
    out_type=jax.ShapeDtypeStruct((NEDGES,), jnp.float32),
    mesh=_mesh,
    scratch_types=[
        [pltpu.VMEM((2, BLK), jnp.int32) for _ in range(R)],
        [pltpu.VMEM((BLK,), jnp.int32) for _ in range(R)],
        [pltpu.VMEM((BLK, EMBED), jnp.float32) for _ in range(R)],
        [pltpu.VMEM((BLK,), jnp.float32) for _ in range(R)],
        pltpu.VMEM((NREL * EMBED,), jnp.float32),
        pltpu.VMEM((BLK * L,), jnp.float32),
        pltpu.VMEM((2 * L,), jnp.float32),
        [pltpu.SemaphoreType.DMA for _ in range(R)],
        [pltpu.SemaphoreType.DMA for _ in range(R)],
        [pltpu.SemaphoreType.DMA for _ in range(R)],
        [pltpu.SemaphoreType.DMA for _ in range(R)],
        [pltpu.SemaphoreType.DMA for _ in range(R)],
    ],
    compiler_params=pltpu.CompilerParams(needs_layout_passes=False),
)
def _sc_scores(table, idx2, rels, rel_flat, params, out,
               idx_v, rid_v, urows, out_v, rel_v, accbuf, par_v,
               semi, semr, semh, sema, semo):
    wid = lax.axis_index("s") * NC + lax.axis_index("c")

    pltpu.sync_copy(rel_flat, rel_v)
    pltpu.sync_copy(params, par_v)
    scale = par_v[pl.ds(0, L)]
    off = par_v[pl.ds(L, L)]
    lanes = lax.iota(jnp.int32, L)

    def jof(k):
        return wid + NW * k

    def valid(k):
        return (k >= 0) & (jof(k) < NBLK)

    def issue_idx(k, b):
        @pl.when(valid(k))
        def _():
            pltpu.async_copy(idx2.at[jof(k)], idx_v[b], semi[b])

    def issue_rid(k, b):
        @pl.when(valid(k))
        def _():
            pltpu.async_copy(rels.at[jof(k)], rid_v[b], semr[b])

    def drain_idx(k, b):
        @pl.when(valid(k))
        def _():
            pltpu.make_async_copy(idx2.at[0], idx_v[b], semi[b]).wait()

    def drain_rid(k, b):
        @pl.when(valid(k))
        def _():
            pltpu.make_async_copy(rels.at[0], rid_v[b], semr[b]).wait()

    def issue_h(k, b):
        @pl.when(valid(k))
        def _():
            pltpu.async_copy(table.at[idx_v[b].at[0]], urows[b], semh[b])

    def drain_h(k, b):
        @pl.when(valid(k))
        def _():
            pltpu.make_async_copy(
                table.at[pl.ds(0, BLK)], urows[b], semh[b]).wait()

    def issue_adds(k, b):
        @pl.when(valid(k))
        def _():
            pltpu.async_copy(table.at[idx_v[b].at[1]], urows[b], sema[b],
                             add=True)

    def drain_adds(k, b):
        @pl.when(valid(k))
        def _():
            pltpu.make_async_copy(
                table.at[pl.ds(0, BLK)], urows[b], sema[b]).wait()

    def drain_out(k, b):
        @pl.when(valid(k))
        def _():
            pltpu.make_async_copy(
                out_v[b], out.at[pl.ds(jof(k) * BLK, BLK)], semo[b]).wait()

    def compute(k, b):
        @pl.when(valid(k))
        def _():
            u = urows[b]
            rid = rid_v[b]

            def grp1(g, c2):
                rb = rid[pl.ds(g * L, L)] * EMBED
                for i in range(L):
                    e = g * L + i
                    rbase = rb[i]
                    acc0 = jnp.zeros((L,), jnp.float32)
                    acc1 = jnp.zeros((L,), jnp.float32)
                    for c in range(EMBED // L):
                        v = (u[e, pl.ds(c * L, L)]
                             + rel_v[pl.ds(rbase + c * L, L)])
                        if c % 2 == 0:
                            acc0 = acc0 + v * v
                        else:
                            acc1 = acc1 + v * v
                    accbuf[pl.ds(e * L, L)] = acc0 + acc1
                return c2

            lax.fori_loop(0, BLK // L, grp1, 0)

            def grp2(g, c2):
                ebase = (g * L + lanes) * L
                s0 = plsc.load_gather(accbuf, [ebase])
                s1 = plsc.load_gather(accbuf, [ebase + 1])
                for c in range(2, L, 2):
                    s0 = s0 + plsc.load_gather(accbuf, [ebase + c])
                    s1 = s1 + plsc.load_gather(accbuf, [ebase + c + 1])
                x = s0 + s1
                i = lax.bitcast_convert_type(x, jnp.int32)
                z = lax.bitcast_convert_type(
                    jnp.int32(0x5F3759DF) - lax.shift_right_logical(i, 1),
                    jnp.float32,
                )
                hx = 0.5 * x
                for _ in range(2):
                    z = z * (1.5 - hx * z * z)
                y = x * z
                out_v[b][pl.ds(g * L, L)] = off - scale * y
                return c2

            lax.fori_loop(0, BLK // L, grp2, 0)
            pltpu.async_copy(out_v[b], out.at[pl.ds(jof(k) * BLK, BLK)],
                             semo[b])

    def step(k, b):
        b1, b2 = (b + 1) % R, (b + 2) % R
        drain_out(k - 2, b1)
        drain_adds(k, b)
        drain_h(k + 1, b1)
        issue_adds(k + 1, b1)
        drain_rid(k, b)
        drain_idx(k + 2, b2)
        issue_h(k + 2, b2)
        issue_idx(k + 3, b)
        compute(k, b)
        issue_rid(k + 3, b)

    issue_idx(0, 0)
    issue_rid(0, 0)
    issue_rid(1, 1)
    issue_rid(2, 2)
    drain_idx(0, 0)
    issue_h(0, 0)
    drain_h(0, 0)
    issue_adds(0, 0)
    issue_idx(1, 1)
    drain_idx(1, 1)
    issue_h(1, 1)
    issue_idx(2, 2)

    ntriples = (KMAX - 1) // R

    def triple(t, carry):
        k = R * t
        step(k, 0)
        step(k + 1, 1)
        step(k + 2, 2)
        return carry

    lax.fori_loop(0, ntriples, triple, 0)
    for k in range(R * ntriples, KMAX):
        step(k, k % R)
    drain_out(KMAX - 2, (KMAX - 2) % R)
    drain_out(KMAX - 1, (KMAX - 1) % R)


def kernel(node_embeddings, edge_index, relation_type, rel_emb, temperature, bias):
    table = jnp.concatenate([node_embeddings, -node_embeddings], axis=0)
    hidx = edge_index[0].astype(jnp.int32)
    tidx = edge_index[1].astype(jnp.int32) + NNODES
    rels = relation_type.astype(jnp.int32).reshape(NBLK, BLK)
    idx2 = jnp.stack(
        [hidx.reshape(NBLK, BLK), tidx.reshape(NBLK, BLK)], axis=1)
    scale = (1.0 / (temperature * math.sqrt(EMBED))).astype(jnp.float32)
    off = (bias / temperature).astype(jnp.float32)
    params = jnp.concatenate(
        [jnp.broadcast_to(scale, (L,)), jnp.broadcast_to(off, (L,))]
    )
    return _sc_scores(table, idx2, rels, rel_emb.reshape(-1), params)
